# baseline (speedup 1.0000x reference)
import functools

import jax
import jax.numpy as jnp
from jax import lax
from jax.experimental import pallas as pl
from jax.experimental.pallas import tpu as pltpu
from jax.experimental.pallas import tpu_sc as plsc

_EMB_DIM = 64
_NB = 16384
_NJ = 20
_B = _NB * _NJ

_info = plsc.get_sparse_core_info()
_NC, _NS = _info.num_cores, _info.num_subcores
_NW = _NC * _NS
_B_PER_W = _B // _NW
_NB_PER_W = _NB // _NW

_CB = 8
_CH = _CB * _NJ
_NSLOT = 3
_N_CH = _B_PER_W // _CH
_N_IT = (_N_CH - 1) // 3


def _gather_body(x_hbm, l_hbm, r_hbm, outl_hbm, outr_hbm,
                 idx_v, lrows, rrows, gsem_l, gsem_r, wsem_l, wsem_r):
    wid = lax.axis_index("s") * _NC + lax.axis_index("c")
    base = wid * _B_PER_W
    base_b = wid * _NB_PER_W

    pltpu.sync_copy(x_hbm.at[pl.ds(base, _B_PER_W)], idx_v)

    def idx_slice(i):
        return idx_v.at[pl.ds(pl.multiple_of(i * _CH, 8), _CH)]

    def fire_g(i, s):
        pltpu.async_copy(l_hbm.at[idx_slice(i)], lrows.at[s], gsem_l.at[s])
        pltpu.async_copy(r_hbm.at[idx_slice(i)], rrows.at[s], gsem_r.at[s])

    def wait_g(i, s):
        pltpu.make_async_copy(
            l_hbm.at[idx_slice(i)], lrows.at[s], gsem_l.at[s]).wait()
        pltpu.make_async_copy(
            r_hbm.at[idx_slice(i)], rrows.at[s], gsem_r.at[s]).wait()

    def fire_w(i, s):
        b0 = base_b + i * _CB
        for k in range(_CB):
            pltpu.async_copy(lrows.at[s].at[pl.ds(k * _NJ, _NJ)],
                             outl_hbm.at[b0 + k], wsem_l.at[s])
            pltpu.async_copy(rrows.at[s].at[pl.ds(k * _NJ, _NJ)],
                             outr_hbm.at[b0 + k], wsem_r.at[s])

    def drain_w(s):
        pltpu.make_async_copy(
            lrows.at[s], l_hbm.at[pl.ds(0, _CH)], wsem_l.at[s]).wait()
        pltpu.make_async_copy(
            rrows.at[s], r_hbm.at[pl.ds(0, _CH)], wsem_r.at[s]).wait()

    fire_g(0, 0)
    fire_g(1, 1)

    def iteration(t, carry):
        for j in range(3):
            i = 3 * t + 1 + j
            wait_g(i - 1, j)
            fire_w(i - 1, j)
            if j == 0:
                @pl.when(t > 0)
                def _():
                    drain_w((j + 2) % 3)
            else:
                drain_w((j + 2) % 3)
            if j == 2:
                @pl.when(t < _N_IT - 1)
                def _():
                    fire_g(i + 1, (j + 2) % 3)
            else:
                fire_g(i + 1, (j + 2) % 3)
        return carry

    lax.fori_loop(0, _N_IT, iteration, 0)

    drain_w(2)
    last = _N_CH - 1
    wait_g(last, last % _NSLOT)
    fire_w(last, last % _NSLOT)
    drain_w(last % _NSLOT)


@jax.jit
def _gather(x_flat, L, R):
    mesh = plsc.VectorSubcoreMesh(core_axis_name="c", subcore_axis_name="s")
    out_type = (
        jax.ShapeDtypeStruct((_NB, _NJ, _EMB_DIM), jnp.float32),
        jax.ShapeDtypeStruct((_NB, _NJ, _EMB_DIM), jnp.float32),
    )
    scratch = [
        pltpu.VMEM((_B_PER_W,), jnp.int32),
        pltpu.VMEM((_NSLOT, _CH, _EMB_DIM), jnp.float32),
        pltpu.VMEM((_NSLOT, _CH, _EMB_DIM), jnp.float32),
        pltpu.SemaphoreType.DMA((_NSLOT,)),
        pltpu.SemaphoreType.DMA((_NSLOT,)),
        pltpu.SemaphoreType.DMA((_NSLOT,)),
        pltpu.SemaphoreType.DMA((_NSLOT,)),
    ]
    fn = functools.partial(
        pl.kernel,
        out_type=out_type,
        mesh=mesh,
        scratch_types=scratch,
        compiler_params=pltpu.CompilerParams(use_tc_tiling_on_sc=False),
    )(_gather_body)
    return fn(x_flat, L, R)


def kernel(x, L, R):
    x_flat = x.reshape(-1)
    return _gather(x_flat, L, R)
